# baseline (speedup 1.0000x reference)
import math

import jax
import jax.numpy as jnp
from jax import lax
from jax.experimental import pallas as pl
from jax.experimental.pallas import tpu as pltpu
from jax.experimental.pallas import tpu_sc as plsc

VOCAB = 1000000
D_MODEL = 64
SCALE = math.sqrt(D_MODEL)

NUM_CORES = 2
NUM_SUBCORES = 16
NW = NUM_CORES * NUM_SUBCORES
LANES = 16

NJ = 200
NS = 4096
SBLK = NS // NW
PAIRW = 2 * D_MODEL

NSLOT = 4
NST = 2

NPAIR = VOCAB // 2
NBLK = VOCAB // 128
NT1 = NBLK // NW + 1


def _relayout_body(tabT_hbm, tail_hbm, tab2_hbm, rbuf, sbuf, tbuf,
                   rsem, wsem):
    wid = lax.axis_index("s") * NUM_CORES + lax.axis_index("c")

    def fire_read(b, slot):
        pltpu.async_copy(tabT_hbm.at[:, pl.ds(b * 128, 128)], rbuf.at[slot],
                         rsem.at[slot])

    def wait_read(b, slot):
        pltpu.make_async_copy(tabT_hbm.at[:, pl.ds(b * 128, 128)],
                              rbuf.at[slot], rsem.at[slot]).wait()

    def fire_write(b, slot):
        pltpu.async_copy(sbuf.at[slot], tab2_hbm.at[pl.ds(b * 64, 64), :],
                         wsem.at[slot])

    def wait_write(b, slot):
        pltpu.make_async_copy(sbuf.at[slot],
                              tab2_hbm.at[pl.ds(b * 64, 64), :],
                              wsem.at[slot]).wait()

    fire_read(wid, 0)

    @pl.loop(0, NT1)
    def _t(t):
        b = wid + NW * t
        slot = lax.rem(t, 2)

        @pl.when(b < NBLK)
        def _full():
            @pl.when(b + NW < NBLK)
            def _prefetch():
                fire_read(b + NW, 1 - slot)

            wait_read(b, slot)

            @pl.when(t >= 2)
            def _drain():
                wait_write(b - 2 * NW, slot)

            slot_v = jnp.full((16,), slot, dtype=jnp.int32)

            @plsc.parallel_loop(0, D_MODEL, unroll=8)
            def _r(r):
                for h in range(2):
                    cv = jnp.full((16,), 2 * r + h, dtype=jnp.int32)
                    for k in range(4):
                        rids = lax.iota(jnp.int32, 16) + (k * LANES)
                        v = plsc.load_gather(rbuf, [slot_v, rids, cv])
                        sbuf[slot, r, pl.ds(64 * h + k * LANES, LANES)] = v

            fire_write(b, slot)

        @pl.when(b == NBLK)
        def _tail():
            pltpu.sync_copy(tail_hbm, tbuf)
            pltpu.sync_copy(tbuf, tab2_hbm.at[pl.ds(NPAIR - 32, 32), :])

    tlast = jnp.where(wid < NBLK - (NT1 - 1) * NW, NT1 - 1, NT1 - 2)
    blast = wid + NW * tlast
    wait_write(blast - NW, lax.rem(tlast - 1, 2))
    wait_write(blast, lax.rem(tlast, 2))


@jax.jit
def _relayout_call(tabT, tail):
    mesh = plsc.VectorSubcoreMesh(core_axis_name="c", subcore_axis_name="s")
    return pl.kernel(
        _relayout_body,
        out_type=jax.ShapeDtypeStruct((NPAIR, 128), jnp.float32),
        mesh=mesh,
        scratch_types=[
            pltpu.VMEM((2, D_MODEL, 128), jnp.float32),
            pltpu.VMEM((2, D_MODEL, 128), jnp.float32),
            pltpu.VMEM((32, 128), jnp.float32),
            pltpu.SemaphoreType.DMA((2,)),
            pltpu.SemaphoreType.DMA((2,)),
        ],
        compiler_params=pltpu.CompilerParams(
            use_tc_tiling_on_sc=True, needs_layout_passes=False),
    )(tabT, tail)


def _emb_body(xT_hbm, tab2_hbm, out_hbm, idx_v, idx2, rows, st, gsem, wsem):
    wid = lax.axis_index("s") * NUM_CORES + lax.axis_index("c")
    s0 = wid * SBLK

    def prep_and_fire(j, slot):
        for k in range(SBLK // LANES):
            sl = pl.ds(k * LANES, LANES)
            idx2[slot, sl] = lax.shift_right_logical(idx_v[j, sl], 1)
        pltpu.async_copy(tab2_hbm.at[idx2.at[slot]], rows.at[slot],
                         gsem.at[slot])

    def wait_gather(slot):
        pltpu.make_async_copy(tab2_hbm.at[idx2.at[slot]], rows.at[slot],
                              gsem.at[slot]).wait()

    def fire_write(p, j):
        pltpu.async_copy(st.at[p], out_hbm.at[j, :, pl.ds(s0, SBLK)],
                         wsem.at[p])

    def wait_write(p, j):
        pltpu.make_async_copy(st.at[p], out_hbm.at[j, :, pl.ds(s0, SBLK)],
                              wsem.at[p]).wait()

    pltpu.sync_copy(xT_hbm.at[:, pl.ds(s0, SBLK)], idx_v)
    for jj in range(NSLOT):
        prep_and_fire(jj, jj)

    @pl.loop(0, NJ)
    def _j(j):
        slot = lax.rem(j, NSLOT)
        p = lax.rem(j, NST)
        wait_gather(slot)

        @pl.when(j >= NST)
        def _drain():
            wait_write(p, j - NST)

        slot_v = jnp.full((16,), slot, dtype=jnp.int32)
        pars = [
            lax.shift_left(
                jnp.bitwise_and(idx_v[j, pl.ds(k * LANES, LANES)], 1), 6)
            for k in range(SBLK // LANES)
        ]

        @plsc.parallel_loop(0, D_MODEL, unroll=8)
        def _d(d):
            dv = jnp.full((16,), d, dtype=jnp.int32)
            for k in range(SBLK // LANES):
                rids = lax.iota(jnp.int32, 16) + (k * LANES)
                v = plsc.load_gather(rows, [slot_v, rids, pars[k] + dv])
                st[p, d, pl.ds(k * LANES, LANES)] = v * SCALE

        fire_write(p, j)

        @pl.when(j + NSLOT < NJ)
        def _refill():
            prep_and_fire(j + NSLOT, slot)

    wait_write(0, NJ - 2)
    wait_write(1, NJ - 1)


@jax.jit
def _emb_call(xT, tab2):
    mesh = plsc.VectorSubcoreMesh(core_axis_name="c", subcore_axis_name="s")
    return pl.kernel(
        _emb_body,
        out_type=jax.ShapeDtypeStruct((NJ, D_MODEL, NS), jnp.float32),
        mesh=mesh,
        scratch_types=[
            pltpu.VMEM((NJ, SBLK), jnp.int32),
            pltpu.VMEM((NSLOT, SBLK), jnp.int32),
            pltpu.VMEM((NSLOT, SBLK, PAIRW), jnp.float32),
            pltpu.VMEM((NST, D_MODEL, SBLK), jnp.float32),
            pltpu.SemaphoreType.DMA((NSLOT,)),
            pltpu.SemaphoreType.DMA((NST,)),
        ],
        compiler_params=pltpu.CompilerParams(
            use_tc_tiling_on_sc=True, needs_layout_passes=False),
    )(xT, tab2)


def kernel(x, table):
    tail = table[VOCAB - 64:].reshape(32, PAIRW)
    tab2 = _relayout_call(table.T, tail)
    out3 = _emb_call(x.T.astype(jnp.int32), tab2)
    return out3.transpose(2, 0, 1)


# speedup vs baseline: 1.1614x; 1.1614x over previous
import math

import jax
import jax.numpy as jnp
from jax import lax
from jax.experimental import pallas as pl
from jax.experimental.pallas import tpu as pltpu
from jax.experimental.pallas import tpu_sc as plsc

VOCAB = 1000000
D_MODEL = 64
SCALE = math.sqrt(D_MODEL)

NUM_CORES = 2
NUM_SUBCORES = 16
NW = NUM_CORES * NUM_SUBCORES
LANES = 16

NJ = 200
NS = 4096
SBLK = NS // NW
PAIRW = 2 * D_MODEL

NSLOT = 4
NST = 2


def _emb_body(xT_hbm, tab2_hbm, out_hbm, idx_v, idx2, rows, st, gsem, wsem):
    wid = lax.axis_index("s") * NUM_CORES + lax.axis_index("c")
    s0 = wid * SBLK

    def prep_and_fire(j, slot):
        for k in range(SBLK // LANES):
            sl = pl.ds(k * LANES, LANES)
            idx2[slot, sl] = lax.shift_right_logical(idx_v[j, sl], 1)
        pltpu.async_copy(tab2_hbm.at[idx2.at[slot]], rows.at[slot],
                         gsem.at[slot])

    def wait_gather(slot):
        pltpu.make_async_copy(tab2_hbm.at[idx2.at[slot]], rows.at[slot],
                              gsem.at[slot]).wait()

    def fire_write(p, j):
        pltpu.async_copy(st.at[p], out_hbm.at[j, :, pl.ds(s0, SBLK)],
                         wsem.at[p])

    def wait_write(p, j):
        pltpu.make_async_copy(st.at[p], out_hbm.at[j, :, pl.ds(s0, SBLK)],
                              wsem.at[p]).wait()

    pltpu.sync_copy(xT_hbm.at[:, pl.ds(s0, SBLK)], idx_v)
    for jj in range(NSLOT):
        prep_and_fire(jj, jj)

    @pl.loop(0, NJ)
    def _j(j):
        slot = lax.rem(j, NSLOT)
        p = lax.rem(j, NST)
        wait_gather(slot)

        @pl.when(j >= NST)
        def _drain():
            wait_write(p, j - NST)

        slot_v = jnp.full((16,), slot, dtype=jnp.int32)
        pars = [
            lax.shift_left(
                jnp.bitwise_and(idx_v[j, pl.ds(k * LANES, LANES)], 1), 6)
            for k in range(SBLK // LANES)
        ]

        @plsc.parallel_loop(0, D_MODEL, unroll=8)
        def _d(d):
            dv = jnp.full((16,), d, dtype=jnp.int32)
            for k in range(SBLK // LANES):
                rids = lax.iota(jnp.int32, 16) + (k * LANES)
                v = plsc.load_gather(rows, [slot_v, rids, pars[k] + dv])
                st[p, d, pl.ds(k * LANES, LANES)] = v * SCALE

        fire_write(p, j)

        @pl.when(j + NSLOT < NJ)
        def _refill():
            prep_and_fire(j + NSLOT, slot)

    wait_write(0, NJ - 2)
    wait_write(1, NJ - 1)


@jax.jit
def _emb_call(xT, tab2):
    mesh = plsc.VectorSubcoreMesh(core_axis_name="c", subcore_axis_name="s")
    return pl.kernel(
        _emb_body,
        out_type=jax.ShapeDtypeStruct((NJ, D_MODEL, NS), jnp.float32),
        mesh=mesh,
        scratch_types=[
            pltpu.VMEM((NJ, SBLK), jnp.int32),
            pltpu.VMEM((NSLOT, SBLK), jnp.int32),
            pltpu.VMEM((NSLOT, SBLK, PAIRW), jnp.float32),
            pltpu.VMEM((NST, D_MODEL, SBLK), jnp.float32),
            pltpu.SemaphoreType.DMA((NSLOT,)),
            pltpu.SemaphoreType.DMA((NST,)),
        ],
        compiler_params=pltpu.CompilerParams(
            use_tc_tiling_on_sc=True, needs_layout_passes=False),
    )(xT, tab2)


def kernel(x, table):
    out3 = _emb_call(x.T.astype(jnp.int32),
                     table.reshape(VOCAB // 2, PAIRW))
    return out3.transpose(2, 0, 1)
